# baseline (speedup 1.0000x reference)
import jax
import jax.numpy as jnp
from jax.experimental import pallas as pl
from jax.experimental.pallas import tpu as pltpu

_NH = 32
_S = 8192
_D = 128
_Q = 16
_SB = 1024
_NSB = _S // _SB


def _kv_update_body(pos_ref, ks_ref, vs_ref, kc_ref, vc_ref, ko_ref, vo_ref):
    s0 = pl.program_id(1) * _SB
    ko_ref[...] = kc_ref[...]
    vo_ref[...] = vc_ref[...]
    for j in range(_Q):
        p = pos_ref[j]
        off = p - s0

        @pl.when(jnp.logical_and(off >= 0, off < _SB))
        def _():
            ko_ref[0, 0, pl.ds(off, 1), :] = ks_ref[0, 0, pl.ds(j, 1), :]
            vo_ref[0, 0, pl.ds(off, 1), :] = vs_ref[0, 0, pl.ds(j, 1), :]


def kernel(key_states, value_states, cache_position, key_cache, value_cache):
    cache_spec = pl.BlockSpec(
        (1, 1, _SB, _D), lambda h, s: (0, h, s, 0))
    states_spec = pl.BlockSpec(
        (1, 1, _Q, _D), lambda h, s: (0, h, 0, 0))
    return pl.pallas_call(
        _kv_update_body,
        grid=(_NH, _NSB),
        out_shape=(
            jax.ShapeDtypeStruct(key_cache.shape, key_cache.dtype),
            jax.ShapeDtypeStruct(value_cache.shape, value_cache.dtype),
        ),
        in_specs=[
            pl.BlockSpec(memory_space=pltpu.SMEM),
            states_spec,
            states_spec,
            cache_spec,
            cache_spec,
        ],
        out_specs=(cache_spec, cache_spec),
        compiler_params=pltpu.CompilerParams(
            dimension_semantics=("arbitrary", "arbitrary"),
        ),
    )(cache_position, key_states, value_states, key_cache, value_cache)


# speedup vs baseline: 1.6060x; 1.6060x over previous
import jax
import jax.numpy as jnp
from jax.experimental import pallas as pl
from jax.experimental.pallas import tpu as pltpu

_NH = 32
_S = 8192
_D = 128
_Q = 16
_SB = 4096
_NSB = _S // _SB


def _kv_update_body(pos_ref, ks_ref, vs_ref, kc_ref, vc_ref, ko_ref, vo_ref):
    s0 = pl.program_id(1) * _SB
    ko_ref[...] = kc_ref[...]
    vo_ref[...] = vc_ref[...]
    for j in range(_Q):
        p = pos_ref[j]
        off = p - s0

        @pl.when(jnp.logical_and(off >= 0, off < _SB))
        def _():
            ko_ref[0, 0, pl.ds(off, 1), :] = ks_ref[0, 0, pl.ds(j, 1), :]
            vo_ref[0, 0, pl.ds(off, 1), :] = vs_ref[0, 0, pl.ds(j, 1), :]


def kernel(key_states, value_states, cache_position, key_cache, value_cache):
    cache_spec = pl.BlockSpec(
        (1, 1, _SB, _D), lambda h, s: (0, h, s, 0))
    states_spec = pl.BlockSpec(
        (1, 1, _Q, _D), lambda h, s: (0, h, 0, 0))
    return pl.pallas_call(
        _kv_update_body,
        grid=(_NH, _NSB),
        out_shape=(
            jax.ShapeDtypeStruct(key_cache.shape, key_cache.dtype),
            jax.ShapeDtypeStruct(value_cache.shape, value_cache.dtype),
        ),
        in_specs=[
            pl.BlockSpec(memory_space=pltpu.SMEM),
            states_spec,
            states_spec,
            cache_spec,
            cache_spec,
        ],
        out_specs=(cache_spec, cache_spec),
        compiler_params=pltpu.CompilerParams(
            dimension_semantics=("arbitrary", "arbitrary"),
        ),
    )(cache_position, key_states, value_states, key_cache, value_cache)


# speedup vs baseline: 1.6333x; 1.0170x over previous
import jax
import jax.numpy as jnp
from jax.experimental import pallas as pl
from jax.experimental.pallas import tpu as pltpu

_NH = 32
_S = 8192
_D = 128
_Q = 16
_SB = 8192
_NSB = _S // _SB


def _kv_update_body(pos_ref, ks_ref, vs_ref, kc_ref, vc_ref, ko_ref, vo_ref):
    s0 = pl.program_id(1) * _SB
    ko_ref[...] = kc_ref[...]
    vo_ref[...] = vc_ref[...]
    for j in range(_Q):
        p = pos_ref[j]
        off = p - s0

        @pl.when(jnp.logical_and(off >= 0, off < _SB))
        def _():
            ko_ref[0, 0, pl.ds(off, 1), :] = ks_ref[0, 0, pl.ds(j, 1), :]
            vo_ref[0, 0, pl.ds(off, 1), :] = vs_ref[0, 0, pl.ds(j, 1), :]


def kernel(key_states, value_states, cache_position, key_cache, value_cache):
    cache_spec = pl.BlockSpec(
        (1, 1, _SB, _D), lambda h, s: (0, h, s, 0))
    states_spec = pl.BlockSpec(
        (1, 1, _Q, _D), lambda h, s: (0, h, 0, 0))
    return pl.pallas_call(
        _kv_update_body,
        grid=(_NH, _NSB),
        out_shape=(
            jax.ShapeDtypeStruct(key_cache.shape, key_cache.dtype),
            jax.ShapeDtypeStruct(value_cache.shape, value_cache.dtype),
        ),
        in_specs=[
            pl.BlockSpec(memory_space=pltpu.SMEM),
            states_spec,
            states_spec,
            cache_spec,
            cache_spec,
        ],
        out_specs=(cache_spec, cache_spec),
        compiler_params=pltpu.CompilerParams(
            dimension_semantics=("arbitrary", "arbitrary"),
        ),
    )(cache_position, key_states, value_states, key_cache, value_cache)
